# Initial kernel scaffold; baseline (speedup 1.0000x reference)
import jax
import jax.numpy as jnp
from jax.experimental import pallas as pl


def kernel(x, mat_idx, valid_gate, rule_outputs, alpha_w):
    raise NotImplementedError("write your pallas kernel here")


# speedup vs baseline: 4.3635x; 4.3635x over previous
import functools

import jax
import jax.numpy as jnp
from jax import lax
from jax.experimental import pallas as pl
from jax.experimental.pallas import tpu as pltpu, tpu_sc as plsc

NC = 2
NS = 16
L = 16
NW = NC * NS


def _meta_predicate_body(T, R, BPW,
                         x_hbm, mat_hbm, gate_hbm, rule_hbm, alpha_hbm,
                         out_hbm,
                         xb, sel2d, gate2d, gidx2d, act2d, outv, a16,
                         sem):
    wid = lax.axis_index("s") * NC + lax.axis_index("c")
    base = wid * BPW
    nchunks = BPW // 128
    npairs = (BPW * R) // 128

    iota = lax.iota(jnp.int32, L)

    def bfly(v, op):
        for k in (1, 2, 4):
            a16[...] = v
            p = plsc.load_gather(a16, [jnp.bitwise_xor(iota, k)])
            v = op(v, p)
        return v

    pltpu.sync_copy(alpha_hbm, a16)
    av = a16[...]
    mx = bfly(av, jnp.maximum)
    ev = jnp.exp(av - mx)
    s8 = bfly(ev, jnp.add)
    wv = ev / s8
    a16[...] = wv

    pltpu.sync_copy(x_hbm.at[pl.ds(base, BPW)], xb)
    descs = []
    for h in range(nchunks):
        idx = xb.at[pl.ds(h * 128, 128)]
        descs.append(pltpu.async_copy(
            mat_hbm.at[idx], sel2d.at[pl.ds(h * 128, 128), :], sem))
        descs.append(pltpu.async_copy(
            gate_hbm.at[idx], gate2d.at[pl.ds(h * 128, 128), :], sem))
    for d in descs:
        d.wait()

    nm = BPW // L
    for r in range(R):
        rsplat = jnp.full((L,), r, jnp.int32)
        for m in range(nm):
            bv = iota + (m * L)
            selv = plsc.load_gather(sel2d, [bv, rsplat])
            gv = selv + (r * T)
            i0 = r * BPW + m * L
            gidx2d[i0 // 128, pl.ds(i0 % 128, L)] = gv

    descs = [pltpu.async_copy(rule_hbm.at[gidx2d.at[j]], act2d.at[j], sem)
             for j in range(npairs)]
    for d in descs:
        d.wait()

    thresh = jnp.float32(0.25)
    for m in range(nm):
        bv = iota + (m * L)
        acc = jnp.zeros((L,), jnp.float32)
        for r in range(R):
            rsplat = jnp.full((L,), r, jnp.int32)
            wrv = plsc.load_gather(a16, [jnp.full((L,), r + 8, jnp.int32)])
            gatev = plsc.load_gather(gate2d, [bv, rsplat])
            i0 = r * BPW + m * L
            actv = act2d[i0 // 128, pl.ds(i0 % 128, L)]
            acc = acc + jnp.where(gatev >= thresh, actv * wrv, 0.0)
        outv[pl.ds(m * L, L)] = acc

    pltpu.sync_copy(outv, out_hbm.at[pl.ds(base, BPW)])


def kernel(x, mat_idx, valid_gate, rule_outputs, alpha_w):
    T, R = mat_idx.shape
    B = x.shape[0]
    BPW = B // NW
    rule_flat = rule_outputs.reshape(R * T)
    alpha16 = jnp.tile(alpha_w, 2)
    x = x.astype(jnp.int32)

    mesh = plsc.VectorSubcoreMesh(core_axis_name="c", subcore_axis_name="s",
                                  num_cores=NC, num_subcores=NS)
    body = functools.partial(_meta_predicate_body, T, R, BPW)
    run = pl.kernel(
        body,
        out_type=jax.ShapeDtypeStruct((B,), jnp.float32),
        mesh=mesh,
        compiler_params=pltpu.CompilerParams(needs_layout_passes=False,
                                             use_tc_tiling_on_sc=False),
        scratch_types=[
            pltpu.VMEM((BPW,), jnp.int32),
            pltpu.VMEM((BPW, R), jnp.int32),
            pltpu.VMEM((BPW, R), jnp.float32),
            pltpu.VMEM((BPW * R // 128, 128), jnp.int32),
            pltpu.VMEM((BPW * R // 128, 128), jnp.float32),
            pltpu.VMEM((BPW,), jnp.float32),
            pltpu.VMEM((16,), jnp.float32),
            pltpu.SemaphoreType.DMA,
        ],
    )
    ret = run(x, mat_idx, valid_gate, rule_flat, alpha16)
    return (ret.reshape(B, 1), jnp.zeros(()))


# speedup vs baseline: 13.6031x; 3.1175x over previous
import functools

import jax
import jax.numpy as jnp
from jax import lax
from jax.experimental import pallas as pl
from jax.experimental.pallas import tpu as pltpu, tpu_sc as plsc

NC = 2
NS = 16
L = 16
NW = NC * NS


def _meta_predicate_body(T, R, BPW,
                         x_hbm, mat_hbm, gate_hbm, rule_hbm, alpha_hbm,
                         out_hbm,
                         xb, selb, gateb, actb, outv, a16, sem):
    wid = lax.axis_index("s") * NC + lax.axis_index("c")
    base = wid * BPW
    nchunks = BPW // 128
    iota = lax.iota(jnp.int32, L)

    def bfly(v, op):
        for k in (1, 2, 4):
            a16[...] = v
            p = plsc.load_gather(a16, [jnp.bitwise_xor(iota, k)])
            v = op(v, p)
        return v

    pltpu.sync_copy(alpha_hbm, a16)
    av = a16[...]
    mx = bfly(av, jnp.maximum)
    ev = jnp.exp(av - mx)
    s8 = bfly(ev, jnp.add)
    wv = ev / s8
    a16[...] = wv

    pltpu.sync_copy(x_hbm.at[pl.ds(base, BPW)], xb)

    sel_d, gate_d = [], []
    for r in range(R):
        for h in range(nchunks):
            idx = xb.at[pl.ds(h * 128, 128)]
            j = r * nchunks + h
            sel_d.append(pltpu.async_copy(
                mat_hbm.at[r].at[idx], selb.at[j], sem))
            gate_d.append(pltpu.async_copy(
                gate_hbm.at[r].at[idx], gateb.at[j], sem))

    act_d = []
    for r in range(R):
        for h in range(nchunks):
            j = r * nchunks + h
            sel_d[j].wait()
            act_d.append(pltpu.async_copy(
                rule_hbm.at[r].at[selb.at[j]], actb.at[j], sem))
    for d in gate_d:
        d.wait()
    for d in act_d:
        d.wait()

    thresh = jnp.float32(0.25)
    nm = BPW // L
    for m in range(nm):
        acc = jnp.zeros((L,), jnp.float32)
        for r in range(R):
            wrv = plsc.load_gather(a16, [jnp.full((L,), r + 8, jnp.int32)])
            i0 = r * BPW + m * L
            j, c = i0 // 128, i0 % 128
            actv = actb[j, pl.ds(c, L)]
            gatev = gateb[j, pl.ds(c, L)]
            acc = acc + jnp.where(gatev >= thresh, actv * wrv, 0.0)
        outv[pl.ds(m * L, L)] = acc

    pltpu.sync_copy(outv, out_hbm.at[pl.ds(base, BPW)])


def kernel(x, mat_idx, valid_gate, rule_outputs, alpha_w):
    T, R = mat_idx.shape
    B = x.shape[0]
    BPW = B // NW
    nrows = BPW * R // 128
    mat_t = mat_idx.T
    gate_t = valid_gate.T
    rule_t = rule_outputs.reshape(R, T)
    alpha16 = jnp.tile(alpha_w, 2)
    x = x.astype(jnp.int32)

    mesh = plsc.VectorSubcoreMesh(core_axis_name="c", subcore_axis_name="s",
                                  num_cores=NC, num_subcores=NS)
    body = functools.partial(_meta_predicate_body, T, R, BPW)
    run = pl.kernel(
        body,
        out_type=jax.ShapeDtypeStruct((B,), jnp.float32),
        mesh=mesh,
        compiler_params=pltpu.CompilerParams(needs_layout_passes=False,
                                             use_tc_tiling_on_sc=False),
        scratch_types=[
            pltpu.VMEM((BPW,), jnp.int32),
            pltpu.VMEM((nrows, 128), jnp.int32),
            pltpu.VMEM((nrows, 128), jnp.float32),
            pltpu.VMEM((nrows, 128), jnp.float32),
            pltpu.VMEM((BPW,), jnp.float32),
            pltpu.VMEM((16,), jnp.float32),
            pltpu.SemaphoreType.DMA,
        ],
    )
    ret = run(x, mat_t, gate_t, rule_t, alpha16)
    return (ret.reshape(B, 1), jnp.zeros(()))


# speedup vs baseline: 14.7074x; 1.0812x over previous
import functools

import jax
import jax.numpy as jnp
from jax import lax
from jax.experimental import pallas as pl
from jax.experimental.pallas import tpu as pltpu, tpu_sc as plsc

NC = 2
NS = 16
L = 16
NW = NC * NS
VBIT = 1 << 30


def _meta_predicate_body(T, R, BPW,
                         x_hbm, packed_hbm, rule_hbm, alpha_hbm,
                         out_hbm,
                         xb, selb, gateb, actb, outv, a16, sem):
    wid = lax.axis_index("s") * NC + lax.axis_index("c")
    base = wid * BPW
    nchunks = BPW // 128
    iota = lax.iota(jnp.int32, L)

    def bfly(v, op):
        for k in (1, 2, 4):
            a16[...] = v
            p = plsc.load_gather(a16, [jnp.bitwise_xor(iota, k)])
            v = op(v, p)
        return v

    pltpu.sync_copy(alpha_hbm, a16)
    av = a16[...]
    mx = bfly(av, jnp.maximum)
    ev = jnp.exp(av - mx)
    s8 = bfly(ev, jnp.add)
    wv = ev / s8
    a16[...] = wv

    pltpu.sync_copy(x_hbm.at[pl.ds(base, BPW)], xb)

    sel_d = []
    for r in range(R):
        for h in range(nchunks):
            idx = xb.at[pl.ds(h * 128, 128)]
            j = r * nchunks + h
            sel_d.append(pltpu.async_copy(
                packed_hbm.at[r].at[idx], selb.at[j], sem))

    act_d = []
    mask = jnp.full((L,), VBIT - 1, jnp.int32)
    one = jnp.full((L,), 1.0, jnp.float32)
    zero = jnp.zeros((L,), jnp.float32)
    for r in range(R):
        for h in range(nchunks):
            j = r * nchunks + h
            sel_d[j].wait()
            for c in range(128 // L):
                pv = selb[j, pl.ds(c * L, L)]
                selb[j, pl.ds(c * L, L)] = pv & mask
                gateb[j, pl.ds(c * L, L)] = jnp.where(pv < VBIT, one, zero)
            act_d.append(pltpu.async_copy(
                rule_hbm.at[r].at[selb.at[j]], actb.at[j], sem))
    for d in act_d:
        d.wait()

    wr_splat = [plsc.load_gather(a16, [jnp.full((L,), r + 8, jnp.int32)])
                for r in range(R)]
    nm = BPW // L
    for m in range(nm):
        acc = jnp.zeros((L,), jnp.float32)
        for r in range(R):
            i0 = r * BPW + m * L
            j, c = i0 // 128, i0 % 128
            actv = actb[j, pl.ds(c, L)]
            gatev = gateb[j, pl.ds(c, L)]
            acc = acc + actv * wr_splat[r] * gatev
        outv[pl.ds(m * L, L)] = acc

    pltpu.sync_copy(outv, out_hbm.at[pl.ds(base, BPW)])


def kernel(x, mat_idx, valid_gate, rule_outputs, alpha_w):
    T, R = mat_idx.shape
    B = x.shape[0]
    BPW = B // NW
    nrows = BPW * R // 128
    mat_t = mat_idx.T
    gate_t = valid_gate.T
    packed = jnp.where(gate_t >= 0.25, mat_t, mat_t | VBIT)
    rule_t = rule_outputs.reshape(R, T)
    alpha16 = jnp.tile(alpha_w, 2)
    x = x.astype(jnp.int32)

    mesh = plsc.VectorSubcoreMesh(core_axis_name="c", subcore_axis_name="s",
                                  num_cores=NC, num_subcores=NS)
    body = functools.partial(_meta_predicate_body, T, R, BPW)
    run = pl.kernel(
        body,
        out_type=jax.ShapeDtypeStruct((B,), jnp.float32),
        mesh=mesh,
        compiler_params=pltpu.CompilerParams(needs_layout_passes=False,
                                             use_tc_tiling_on_sc=False),
        scratch_types=[
            pltpu.VMEM((BPW,), jnp.int32),
            pltpu.VMEM((nrows, 128), jnp.int32),
            pltpu.VMEM((nrows, 128), jnp.float32),
            pltpu.VMEM((nrows, 128), jnp.float32),
            pltpu.VMEM((BPW,), jnp.float32),
            pltpu.VMEM((16,), jnp.float32),
            pltpu.SemaphoreType.DMA,
        ],
    )
    ret = run(x, packed, rule_t, alpha16)
    return (ret.reshape(B, 1), jnp.zeros(()))
